# baseline (speedup 1.0000x reference)
import dataclasses
import functools

import jax
import jax.numpy as jnp
import numpy as np
from jax import lax
from jax.experimental import pallas as pl
from jax.experimental.pallas import tpu as pltpu
from jax.experimental.pallas import tpu_sc as plsc

NT = 16
NBITS = 11
NB = 1 << NBITS
RD = NB // NT
W = 8192
NSEG = 3
DUMPN = 1024
INT_MIN = np.int32(-(1 << 31))


def _transform(x_f32):
    b = plsc.bitcast(x_f32, jnp.int32)
    m = lax.shift_right_arithmetic(b, 31)
    return b ^ (m | INT_MIN)


def _untransform(key_i32):
    t = lax.shift_right_arithmetic(key_i32, 31)
    return plsc.bitcast(key_i32 ^ (INT_MIN | ~t), jnp.float32)


def _digit(key_i32, pass_idx):
    u = plsc.bitcast(key_i32, jnp.uint32)
    if pass_idx:
        u = lax.shift_right_logical(u, np.uint32(NBITS * pass_idx))
    return (u & np.uint32(NB - 1)).astype(jnp.int32)


def _make_pass(B, N, pass_idx):
    CHUNK = N // NT
    NWIN = CHUNK // W
    assert CHUNK % W == 0
    SEG = ((N + NSEG - 1) // NSEG + 255) & ~255
    seg_sizes = [SEG] * (NSEG - 1) + [N - (NSEG - 1) * SEG]
    assert all(s > 0 and s % 256 == 0 for s in seg_sizes)

    mesh = plsc.VectorSubcoreMesh(core_axis_name="c", subcore_axis_name="s")
    cp = pltpu.CompilerParams()
    if "needs_layout_passes" in pltpu.CompilerParams.__dataclass_fields__:
        cp = dataclasses.replace(cp, needs_layout_passes=False)

    @functools.partial(
        pl.kernel,
        mesh=mesh,
        compiler_params=cp,
        out_type=jax.ShapeDtypeStruct((B * N,), jnp.float32),
        scratch_types=[
            pltpu.VMEM((NB,), jnp.int32),
            pltpu.VMEM((W,), jnp.float32),
            pltpu.VMEM((W,), jnp.float32),
            pltpu.VMEM((W,), jnp.int32),
            pltpu.VMEM((RD,), jnp.int32),
            pltpu.VMEM((NT * NT,), jnp.int32),
            pltpu.VMEM((NT,), jnp.int32),
            pltpu.VMEM_SHARED((NT, NB), jnp.int32),
            pltpu.VMEM_SHARED((NT * NT,), jnp.int32),
            pltpu.VMEM_SHARED((SEG + DUMPN,), jnp.float32),
            pltpu.SemaphoreType.DMA,
        ],
    )
    def pass_kernel(src, dst,
                    hist, win, stage, pos_buf,
                    tot, ttile, sb16, grid, totals_sp, seg_buf, sem):
        cid = lax.axis_index("c")
        sid = lax.axis_index("s")
        lane = lax.iota(jnp.int32, 16)
        zeros16 = jnp.zeros((16,), jnp.int32)
        ones = jnp.ones((16,), jnp.int32)
        batch_base = cid * N
        chunk_addr = batch_base + sid * CHUNK

        @pl.loop(0, NB, step=16)
        def _(o):
            hist[pl.ds(o, 16)] = zeros16

        @pl.loop(0, NWIN)
        def _(w):
            pltpu.sync_copy(src.at[pl.ds(chunk_addr + w * W, W)], win)

            @pl.loop(0, W, step=16)
            def _(j):
                x = win[pl.ds(j, 16)]
                key = _transform(x) if pass_idx == 0 \
                    else plsc.bitcast(x, jnp.int32)
                plsc.addupdate_scatter(hist, [_digit(key, pass_idx)], ones)

        pltpu.sync_copy(hist, grid.at[sid])
        plsc.subcore_barrier()

        for t in range(NT):
            pltpu.sync_copy(
                grid.at[t, pl.ds(sid * RD, RD)],
                hist.at[pl.ds(t * RD, RD)],
            )

        @pl.loop(0, RD, step=16)
        def _(j):
            acc = zeros16
            for t in range(NT):
                sl = pl.ds(t * RD + j, 16)
                old = hist[sl]
                hist[sl] = acc
                acc = acc + old
            tot[pl.ds(j, 16)] = acc

        def scan_body(j, carry):
            v = tot[pl.ds(j * 16, 16)]
            cs = plsc.cumsum(v)
            tot[pl.ds(j * 16, 16)] = cs - v + carry
            return carry + jnp.sum(v)

        grand = lax.fori_loop(0, RD // 16, scan_body, jnp.int32(0))

        sb16[...] = jnp.broadcast_to(grand, (16,))
        pltpu.sync_copy(sb16, totals_sp.at[pl.ds(sid * 16, 16)])
        plsc.subcore_barrier()
        pltpu.sync_copy(totals_sp, ttile)
        tvec = plsc.load_gather(ttile, [lane * 16])
        cst = plsc.cumsum(tvec)
        sb16[...] = cst - tvec
        slice_base = plsc.load_gather(sb16, [jnp.broadcast_to(sid, (16,))])

        @pl.loop(0, RD, step=16)
        def _(j):
            add = slice_base + tot[pl.ds(j, 16)]
            for t in range(NT):
                sl = pl.ds(t * RD + j, 16)
                hist[sl] = hist[sl] + add

        for t in range(NT):
            pltpu.sync_copy(
                hist.at[pl.ds(t * RD, RD)],
                grid.at[t, pl.ds(sid * RD, RD)],
            )
        plsc.subcore_barrier()
        pltpu.sync_copy(grid.at[sid], hist)
        plsc.subcore_barrier()

        for seg in range(NSEG):
            seg_start = seg * SEG
            seg_size = seg_sizes[seg]
            if seg:
                pltpu.sync_copy(grid.at[sid], hist)

            @pl.loop(0, NWIN)
            def _(w):
                pltpu.sync_copy(src.at[pl.ds(chunk_addr + w * W, W)], win)

                @pl.loop(0, W, step=16)
                def _(j):
                    x = win[pl.ds(j, 16)]
                    key = _transform(x) if pass_idx == 0 \
                        else plsc.bitcast(x, jnp.int32)
                    d = _digit(key, pass_idx)
                    cnt, last = plsc.scan_count(d)
                    base = plsc.load_gather(hist, [d])
                    pos = base + cnt - 1
                    plsc.store_scatter(hist, [d], base + cnt, mask=last)
                    rel = pos - seg_start
                    inseg = plsc.bitcast(rel, jnp.uint32) < np.uint32(seg_size)
                    dump = SEG + (j & (DUMPN - 16)) + lane
                    pos_buf[pl.ds(j, 16)] = jnp.where(inseg, rel, dump)
                    if pass_idx == 2:
                        stage[pl.ds(j, 16)] = _untransform(key)
                    else:
                        stage[pl.ds(j, 16)] = plsc.bitcast(key, jnp.float32)

                pltpu.async_copy(stage, seg_buf.at[pos_buf], sem).wait()

            plsc.subcore_barrier()
            ds_size = seg_size // NT
            n_full = ds_size // W
            rem = ds_size - n_full * W
            base_sp = sid * ds_size
            base_hbm = batch_base + seg_start + sid * ds_size

            @pl.loop(0, n_full)
            def _(b):
                pltpu.sync_copy(seg_buf.at[pl.ds(base_sp + b * W, W)], win)
                pltpu.sync_copy(win, dst.at[pl.ds(base_hbm + b * W, W)])

            if rem:
                pltpu.sync_copy(
                    seg_buf.at[pl.ds(base_sp + n_full * W, rem)],
                    win.at[pl.ds(0, rem)])
                pltpu.sync_copy(
                    win.at[pl.ds(0, rem)],
                    dst.at[pl.ds(base_hbm + n_full * W, rem)])
            plsc.subcore_barrier()

    return pass_kernel


def _make_sort(B, N):
    passes = [_make_pass(B, N, p) for p in range(3)]

    def sort(v_flat):
        t0 = passes[0](v_flat)
        t1 = passes[1](t0)
        return passes[2](t1), t1

    return sort


def kernel(q, k, v):
    B = v.shape[0]
    N = v.size // B
    v_flat = v.reshape(B * N)
    out_flat, _ = _make_sort(B, N)(v_flat)
    return out_flat.reshape(v.shape)


# speedup vs baseline: 1.0950x; 1.0950x over previous
import dataclasses
import functools

import jax
import jax.numpy as jnp
import numpy as np
from jax import lax
from jax.experimental import pallas as pl
from jax.experimental.pallas import tpu as pltpu
from jax.experimental.pallas import tpu_sc as plsc

NT = 16
NBITS = 11
NB = 1 << NBITS
RD = NB // NT
W = 4096
NSEG = 3
DUMPN = 1024
INT_MIN = np.int32(-(1 << 31))


def _transform(x_f32):
    b = plsc.bitcast(x_f32, jnp.int32)
    m = lax.shift_right_arithmetic(b, 31)
    return b ^ (m | INT_MIN)


def _untransform(key_i32):
    t = lax.shift_right_arithmetic(key_i32, 31)
    return plsc.bitcast(key_i32 ^ (INT_MIN | ~t), jnp.float32)


def _digit(key_i32, pass_idx):
    u = plsc.bitcast(key_i32, jnp.uint32)
    if pass_idx:
        u = lax.shift_right_logical(u, np.uint32(NBITS * pass_idx))
    return (u & np.uint32(NB - 1)).astype(jnp.int32)


def _make_pass(B, N, pass_idx):
    CHUNK = N // NT
    NWIN = CHUNK // W
    assert CHUNK % W == 0
    SEG = ((N + NSEG - 1) // NSEG + 255) & ~255
    seg_sizes = [SEG] * (NSEG - 1) + [N - (NSEG - 1) * SEG]
    assert all(s > 0 and s % 256 == 0 for s in seg_sizes)

    mesh = plsc.VectorSubcoreMesh(core_axis_name="c", subcore_axis_name="s")
    cp = pltpu.CompilerParams()
    if "needs_layout_passes" in pltpu.CompilerParams.__dataclass_fields__:
        cp = dataclasses.replace(cp, needs_layout_passes=False)

    @functools.partial(
        pl.kernel,
        mesh=mesh,
        compiler_params=cp,
        out_type=jax.ShapeDtypeStruct((B * N,), jnp.float32),
        scratch_types=[
            pltpu.VMEM((NB,), jnp.int32),
            pltpu.VMEM((W,), jnp.float32),
            pltpu.VMEM((W,), jnp.float32),
            pltpu.VMEM((W,), jnp.float32),
            pltpu.VMEM((W,), jnp.float32),
            pltpu.VMEM((W,), jnp.int32),
            pltpu.VMEM((W,), jnp.int32),
            pltpu.VMEM((RD,), jnp.int32),
            pltpu.VMEM((NT * NT,), jnp.int32),
            pltpu.VMEM((NT,), jnp.int32),
            pltpu.VMEM_SHARED((NT, NB), jnp.int32),
            pltpu.VMEM_SHARED((NT * NT,), jnp.int32),
            pltpu.VMEM_SHARED((SEG + DUMPN,), jnp.float32),
            pltpu.SemaphoreType.DMA,
            pltpu.SemaphoreType.DMA,
            pltpu.SemaphoreType.DMA,
            pltpu.SemaphoreType.DMA,
        ],
    )
    def pass_kernel(src, dst,
                    hist, win_a, win_b, stage_a, stage_b, pos_a, pos_b,
                    tot, ttile, sb16, grid, totals_sp, seg_buf,
                    sem_a, sem_b, sem_sa, sem_sb):
        cid = lax.axis_index("c")
        sid = lax.axis_index("s")
        lane = lax.iota(jnp.int32, 16)
        zeros16 = jnp.zeros((16,), jnp.int32)
        ones = jnp.ones((16,), jnp.int32)
        batch_base = cid * N
        chunk_addr = batch_base + sid * CHUNK

        def hist_window(win):
            @pl.loop(0, W, step=16)
            def _(j):
                x = win[pl.ds(j, 16)]
                key = _transform(x) if pass_idx == 0 \
                    else plsc.bitcast(x, jnp.int32)
                plsc.addupdate_scatter(hist, [_digit(key, pass_idx)], ones)

        @pl.loop(0, NB, step=16)
        def _(o):
            hist[pl.ds(o, 16)] = zeros16

        @pl.loop(0, NWIN, step=2)
        def _(w):
            a_in = pltpu.async_copy(
                src.at[pl.ds(chunk_addr + w * W, W)], win_a, sem_a)
            b_in = pltpu.async_copy(
                src.at[pl.ds(chunk_addr + (w + 1) * W, W)], win_b, sem_b)
            a_in.wait()
            hist_window(win_a)
            b_in.wait()
            hist_window(win_b)

        pltpu.sync_copy(hist, grid.at[sid])
        plsc.subcore_barrier()

        for t in range(NT):
            pltpu.sync_copy(
                grid.at[t, pl.ds(sid * RD, RD)],
                hist.at[pl.ds(t * RD, RD)],
            )

        @pl.loop(0, RD, step=16)
        def _(j):
            acc = zeros16
            for t in range(NT):
                sl = pl.ds(t * RD + j, 16)
                old = hist[sl]
                hist[sl] = acc
                acc = acc + old
            tot[pl.ds(j, 16)] = acc

        def scan_body(j, carry):
            v = tot[pl.ds(j * 16, 16)]
            cs = plsc.cumsum(v)
            tot[pl.ds(j * 16, 16)] = cs - v + carry
            return carry + jnp.sum(v)

        grand = lax.fori_loop(0, RD // 16, scan_body, jnp.int32(0))

        sb16[...] = jnp.broadcast_to(grand, (16,))
        pltpu.sync_copy(sb16, totals_sp.at[pl.ds(sid * 16, 16)])
        plsc.subcore_barrier()
        pltpu.sync_copy(totals_sp, ttile)
        tvec = plsc.load_gather(ttile, [lane * 16])
        cst = plsc.cumsum(tvec)
        sb16[...] = cst - tvec
        slice_base = plsc.load_gather(sb16, [jnp.broadcast_to(sid, (16,))])

        @pl.loop(0, RD, step=16)
        def _(j):
            add = slice_base + tot[pl.ds(j, 16)]
            for t in range(NT):
                sl = pl.ds(t * RD + j, 16)
                hist[sl] = hist[sl] + add

        for t in range(NT):
            pltpu.sync_copy(
                hist.at[pl.ds(t * RD, RD)],
                grid.at[t, pl.ds(sid * RD, RD)],
            )
        plsc.subcore_barrier()
        pltpu.sync_copy(grid.at[sid], hist)
        plsc.subcore_barrier()

        for seg in range(NSEG):
            seg_start = seg * SEG
            seg_size = seg_sizes[seg]
            if seg:
                pltpu.sync_copy(grid.at[sid], hist)

            def rank_window(win, stage, pos_buf):
                @pl.loop(0, W, step=16)
                def _(j):
                    x = win[pl.ds(j, 16)]
                    key = _transform(x) if pass_idx == 0 \
                        else plsc.bitcast(x, jnp.int32)
                    d = _digit(key, pass_idx)
                    cnt, last = plsc.scan_count(d)
                    base = plsc.load_gather(hist, [d])
                    pos = base + cnt - 1
                    plsc.store_scatter(hist, [d], base + cnt, mask=last)
                    rel = pos - seg_start
                    inseg = plsc.bitcast(rel, jnp.uint32) < np.uint32(seg_size)
                    dump = SEG + (j & (DUMPN - 16)) + lane
                    pos_buf[pl.ds(j, 16)] = jnp.where(inseg, rel, dump)
                    if pass_idx == 2:
                        stage[pl.ds(j, 16)] = _untransform(key)
                    else:
                        stage[pl.ds(j, 16)] = plsc.bitcast(key, jnp.float32)

            @pl.loop(0, NWIN, step=2)
            def _(w):
                a_in = pltpu.async_copy(
                    src.at[pl.ds(chunk_addr + w * W, W)], win_a, sem_a)
                b_in = pltpu.async_copy(
                    src.at[pl.ds(chunk_addr + (w + 1) * W, W)], win_b, sem_b)
                a_in.wait()
                rank_window(win_a, stage_a, pos_a)
                a_sc = pltpu.async_copy(stage_a, seg_buf.at[pos_a], sem_sa)
                b_in.wait()
                rank_window(win_b, stage_b, pos_b)
                b_sc = pltpu.async_copy(stage_b, seg_buf.at[pos_b], sem_sb)
                a_sc.wait()
                b_sc.wait()

            plsc.subcore_barrier()
            ds_size = seg_size // NT
            n_full = ds_size // W
            rem = ds_size - n_full * W
            base_sp = sid * ds_size
            base_hbm = batch_base + seg_start + sid * ds_size

            @pl.loop(0, n_full)
            def _(b):
                pltpu.sync_copy(seg_buf.at[pl.ds(base_sp + b * W, W)], win_a)
                pltpu.sync_copy(win_a, dst.at[pl.ds(base_hbm + b * W, W)])

            if rem:
                pltpu.sync_copy(
                    seg_buf.at[pl.ds(base_sp + n_full * W, rem)],
                    win_a.at[pl.ds(0, rem)])
                pltpu.sync_copy(
                    win_a.at[pl.ds(0, rem)],
                    dst.at[pl.ds(base_hbm + n_full * W, rem)])
            plsc.subcore_barrier()

    return pass_kernel


def _make_sort(B, N):
    passes = [_make_pass(B, N, p) for p in range(3)]

    def sort(v_flat):
        t0 = passes[0](v_flat)
        t1 = passes[1](t0)
        return passes[2](t1), t1

    return sort


def kernel(q, k, v):
    B = v.shape[0]
    N = v.size // B
    v_flat = v.reshape(B * N)
    out_flat, _ = _make_sort(B, N)(v_flat)
    return out_flat.reshape(v.shape)


# speedup vs baseline: 1.5249x; 1.3927x over previous
import dataclasses
import functools

import jax
import jax.numpy as jnp
import numpy as np
from jax import lax
from jax.experimental import pallas as pl
from jax.experimental.pallas import tpu as pltpu
from jax.experimental.pallas import tpu_sc as plsc

NT = 16
NBITS = 11
NB = 1 << NBITS
RD = NB // NT
W = 4096
NSEG = 3
DUMPN = 1024
INT_MIN = np.int32(-(1 << 31))


def _transform(x_f32):
    b = plsc.bitcast(x_f32, jnp.int32)
    m = lax.shift_right_arithmetic(b, 31)
    return b ^ (m | INT_MIN)


def _untransform(key_i32):
    t = lax.shift_right_arithmetic(key_i32, 31)
    return plsc.bitcast(key_i32 ^ (INT_MIN | ~t), jnp.float32)


def _digit(key_i32, pass_idx):
    u = plsc.bitcast(key_i32, jnp.uint32)
    if pass_idx:
        u = lax.shift_right_logical(u, np.uint32(NBITS * pass_idx))
    return (u & np.uint32(NB - 1)).astype(jnp.int32)


def _make_pass(B, N, pass_idx):
    CHUNK = N // NT
    NWIN = CHUNK // W
    assert CHUNK % W == 0
    SEG = ((N + NSEG - 1) // NSEG + 255) & ~255
    seg_sizes = [SEG] * (NSEG - 1) + [N - (NSEG - 1) * SEG]
    assert all(s > 0 and s % 256 == 0 for s in seg_sizes)

    mesh = plsc.VectorSubcoreMesh(core_axis_name="c", subcore_axis_name="s")
    cp = pltpu.CompilerParams()
    if "needs_layout_passes" in pltpu.CompilerParams.__dataclass_fields__:
        cp = dataclasses.replace(cp, needs_layout_passes=False)

    @functools.partial(
        pl.kernel,
        mesh=mesh,
        compiler_params=cp,
        out_type=[
            jax.ShapeDtypeStruct((B * N,), jnp.float32),
            jax.ShapeDtypeStruct((B * N,), jnp.int32),
        ],
        scratch_types=[
            pltpu.VMEM((NB,), jnp.int32),
            pltpu.VMEM((W,), jnp.float32),
            pltpu.VMEM((W,), jnp.float32),
            pltpu.VMEM((W,), jnp.float32),
            pltpu.VMEM((W,), jnp.float32),
            pltpu.VMEM((W,), jnp.int32),
            pltpu.VMEM((W,), jnp.int32),
            pltpu.VMEM((W,), jnp.int32),
            pltpu.VMEM((W,), jnp.int32),
            pltpu.VMEM((RD,), jnp.int32),
            pltpu.VMEM((NT * NT,), jnp.int32),
            pltpu.VMEM((NT,), jnp.int32),
            pltpu.VMEM_SHARED((NT, NB), jnp.int32),
            pltpu.VMEM_SHARED((NT * NT,), jnp.int32),
            pltpu.VMEM_SHARED((SEG + DUMPN,), jnp.float32),
            pltpu.SemaphoreType.DMA,
            pltpu.SemaphoreType.DMA,
            pltpu.SemaphoreType.DMA,
            pltpu.SemaphoreType.DMA,
            pltpu.SemaphoreType.DMA,
            pltpu.SemaphoreType.DMA,
        ],
    )
    def pass_kernel(src, dst, pos_hbm,
                    hist, win_a, win_b, stage_a, stage_b, idx_a, idx_b,
                    pos_a, pos_b,
                    tot, ttile, sb16, grid, totals_sp, seg_buf,
                    sem_a, sem_b, sem_sa, sem_sb, sem_pa, sem_pb):
        cid = lax.axis_index("c")
        sid = lax.axis_index("s")
        lane = lax.iota(jnp.int32, 16)
        zeros16 = jnp.zeros((16,), jnp.int32)
        ones = jnp.ones((16,), jnp.int32)
        batch_base = cid * N
        chunk_addr = batch_base + sid * CHUNK

        def hist_window(win):
            @pl.loop(0, W, step=16)
            def _(j):
                x = win[pl.ds(j, 16)]
                key = _transform(x) if pass_idx == 0 \
                    else plsc.bitcast(x, jnp.int32)
                plsc.addupdate_scatter(hist, [_digit(key, pass_idx)], ones)

        @pl.loop(0, NB, step=16)
        def _(o):
            hist[pl.ds(o, 16)] = zeros16

        @pl.loop(0, NWIN, step=2)
        def _(w):
            a_in = pltpu.async_copy(
                src.at[pl.ds(chunk_addr + w * W, W)], win_a, sem_a)
            b_in = pltpu.async_copy(
                src.at[pl.ds(chunk_addr + (w + 1) * W, W)], win_b, sem_b)
            a_in.wait()
            hist_window(win_a)
            b_in.wait()
            hist_window(win_b)

        pltpu.sync_copy(hist, grid.at[sid])
        plsc.subcore_barrier()

        for t in range(NT):
            pltpu.sync_copy(
                grid.at[t, pl.ds(sid * RD, RD)],
                hist.at[pl.ds(t * RD, RD)],
            )

        @pl.loop(0, RD, step=16)
        def _(j):
            acc = zeros16
            for t in range(NT):
                sl = pl.ds(t * RD + j, 16)
                old = hist[sl]
                hist[sl] = acc
                acc = acc + old
            tot[pl.ds(j, 16)] = acc

        def scan_body(j, carry):
            v = tot[pl.ds(j * 16, 16)]
            cs = plsc.cumsum(v)
            tot[pl.ds(j * 16, 16)] = cs - v + carry
            return carry + jnp.sum(v)

        grand = lax.fori_loop(0, RD // 16, scan_body, jnp.int32(0))

        sb16[...] = jnp.broadcast_to(grand, (16,))
        pltpu.sync_copy(sb16, totals_sp.at[pl.ds(sid * 16, 16)])
        plsc.subcore_barrier()
        pltpu.sync_copy(totals_sp, ttile)
        tvec = plsc.load_gather(ttile, [lane * 16])
        cst = plsc.cumsum(tvec)
        sb16[...] = cst - tvec
        slice_base = plsc.load_gather(sb16, [jnp.broadcast_to(sid, (16,))])

        @pl.loop(0, RD, step=16)
        def _(j):
            add = slice_base + tot[pl.ds(j, 16)]
            for t in range(NT):
                sl = pl.ds(t * RD + j, 16)
                hist[sl] = hist[sl] + add

        for t in range(NT):
            pltpu.sync_copy(
                hist.at[pl.ds(t * RD, RD)],
                grid.at[t, pl.ds(sid * RD, RD)],
            )
        plsc.subcore_barrier()
        pltpu.sync_copy(grid.at[sid], hist)
        plsc.subcore_barrier()

        for seg in range(NSEG):
            seg_start = seg * SEG
            seg_size = seg_sizes[seg]
            def payload(key):
                if pass_idx == 2:
                    return _untransform(key)
                return plsc.bitcast(key, jnp.float32)

            def seg_idx(pos, j):
                rel = pos - seg_start
                inseg = plsc.bitcast(rel, jnp.uint32) < np.uint32(seg_size)
                dump = SEG + (j & (DUMPN - 16)) + lane
                return jnp.where(inseg, rel, dump)

            def rank_window(win, stage, idx, posr):
                @pl.loop(0, W, step=16)
                def _(j):
                    x = win[pl.ds(j, 16)]
                    key = _transform(x) if pass_idx == 0 \
                        else plsc.bitcast(x, jnp.int32)
                    d = _digit(key, pass_idx)
                    cnt, last = plsc.scan_count(d)
                    base = plsc.load_gather(hist, [d])
                    pos = base + cnt - 1
                    plsc.store_scatter(hist, [d], base + cnt, mask=last)
                    posr[pl.ds(j, 16)] = pos
                    idx[pl.ds(j, 16)] = seg_idx(pos, j)
                    stage[pl.ds(j, 16)] = payload(key)

            def light_window(win, stage, idx, posr):
                @pl.loop(0, W, step=16)
                def _(j):
                    x = win[pl.ds(j, 16)]
                    key = _transform(x) if pass_idx == 0 \
                        else plsc.bitcast(x, jnp.int32)
                    pos = posr[pl.ds(j, 16)]
                    idx[pl.ds(j, 16)] = seg_idx(pos, j)
                    stage[pl.ds(j, 16)] = payload(key)

            @pl.loop(0, NWIN, step=2)
            def _(w):
                sl_a = pl.ds(chunk_addr + w * W, W)
                sl_b = pl.ds(chunk_addr + (w + 1) * W, W)
                a_in = pltpu.async_copy(src.at[sl_a], win_a, sem_a)
                b_in = pltpu.async_copy(src.at[sl_b], win_b, sem_b)
                if seg == 0:
                    a_in.wait()
                    rank_window(win_a, stage_a, idx_a, pos_a)
                    a_sc = pltpu.async_copy(stage_a, seg_buf.at[idx_a], sem_sa)
                    a_ps = pltpu.async_copy(pos_a, pos_hbm.at[sl_a], sem_pa)
                    b_in.wait()
                    rank_window(win_b, stage_b, idx_b, pos_b)
                    b_sc = pltpu.async_copy(stage_b, seg_buf.at[idx_b], sem_sb)
                    b_ps = pltpu.async_copy(pos_b, pos_hbm.at[sl_b], sem_pb)
                    a_sc.wait(); a_ps.wait(); b_sc.wait(); b_ps.wait()
                else:
                    a_pi = pltpu.async_copy(pos_hbm.at[sl_a], pos_a, sem_pa)
                    b_pi = pltpu.async_copy(pos_hbm.at[sl_b], pos_b, sem_pb)
                    a_in.wait(); a_pi.wait()
                    light_window(win_a, stage_a, idx_a, pos_a)
                    a_sc = pltpu.async_copy(stage_a, seg_buf.at[idx_a], sem_sa)
                    b_in.wait(); b_pi.wait()
                    light_window(win_b, stage_b, idx_b, pos_b)
                    b_sc = pltpu.async_copy(stage_b, seg_buf.at[idx_b], sem_sb)
                    a_sc.wait(); b_sc.wait()

            plsc.subcore_barrier()
            ds_size = seg_size // NT
            n_full = ds_size // W
            rem = ds_size - n_full * W
            base_sp = sid * ds_size
            base_hbm = batch_base + seg_start + sid * ds_size

            @pl.loop(0, n_full)
            def _(b):
                pltpu.sync_copy(seg_buf.at[pl.ds(base_sp + b * W, W)], win_a)
                pltpu.sync_copy(win_a, dst.at[pl.ds(base_hbm + b * W, W)])

            if rem:
                pltpu.sync_copy(
                    seg_buf.at[pl.ds(base_sp + n_full * W, rem)],
                    win_a.at[pl.ds(0, rem)])
                pltpu.sync_copy(
                    win_a.at[pl.ds(0, rem)],
                    dst.at[pl.ds(base_hbm + n_full * W, rem)])
            plsc.subcore_barrier()

    return pass_kernel


def _make_sort(B, N):
    passes = [_make_pass(B, N, p) for p in range(3)]

    def sort(v_flat):
        t0, _ = passes[0](v_flat)
        t1, _ = passes[1](t0)
        out, _ = passes[2](t1)
        return out, t1

    return sort


def kernel(q, k, v):
    B = v.shape[0]
    N = v.size // B
    v_flat = v.reshape(B * N)
    out_flat, _ = _make_sort(B, N)(v_flat)
    return out_flat.reshape(v.shape)
